# Initial kernel scaffold; baseline (speedup 1.0000x reference)
import jax
import jax.numpy as jnp
from jax.experimental import pallas as pl


def kernel(block_outputs, pos_edge_index, neg_edge_index):
    raise NotImplementedError("write your pallas kernel here")


# speedup vs baseline: 4.8597x; 4.8597x over previous
import functools
import math

import jax
import jax.numpy as jnp
from jax import lax
from jax.experimental import pallas as pl
from jax.experimental.pallas import tpu as pltpu
from jax.experimental.pallas import tpu_sc as plsc

_N_NODES = 10000
_E = 320000
_D = 128
_PROB = _E / (_N_NODES ** 2 - _N_NODES) * 2.0
_EPS = -math.log(1.0 - _PROB)

_NC = 2
_NS = 16
_NW = _NC * _NS
_EPT = _E // _NW
_B = 80
_NB = _EPT // _B
_G = _B // 16
_L = 16


def _sc_body(table_hbm, ps_hbm, pd_hbm, ns_hbm, nd_hbm, pout_hbm, nout_hbm,
             sidx, didx, srows, trows, pbuf, accbuf, sem0, sem1):
    wid = lax.axis_index("s") * _NC + lax.axis_index("c")
    base = wid * _EPT

    pltpu.sync_copy(ps_hbm.at[pl.ds(base, _EPT)], sidx)
    pltpu.sync_copy(pd_hbm.at[pl.ds(base, _EPT)], didx)

    def pos_batch(b, carry):
        cs = pltpu.async_copy(
            table_hbm.at[sidx.at[pl.ds(b * _B, _B)]], srows, sem0)
        ct = pltpu.async_copy(
            table_hbm.at[didx.at[pl.ds(b * _B, _B)]], trows, sem1)
        cs.wait()
        ct.wait()

        def group(g, c2):
            for e in range(16):
                row = g * 16 + e
                acc = srows[row, pl.ds(0, _L)] * trows[row, pl.ds(0, _L)]
                for j in range(1, _D // _L):
                    acc = acc + (srows[row, pl.ds(j * _L, _L)]
                                 * trows[row, pl.ds(j * _L, _L)])
                pbuf[pl.ds((g * 16 + e) * _L, _L)] = acc
            return c2

        lax.fori_loop(0, _G, group, 0)
        pltpu.sync_copy(pbuf, pout_hbm.at[pl.ds((base + b * _B) * _L, _B * _L)])
        return carry

    lax.fori_loop(0, _NB, pos_batch, 0)

    pltpu.sync_copy(ns_hbm.at[pl.ds(base, _EPT)], sidx)
    pltpu.sync_copy(nd_hbm.at[pl.ds(base, _EPT)], didx)

    def neg_batch(b, acc):
        cs = pltpu.async_copy(
            table_hbm.at[sidx.at[pl.ds(b * _B, _B)]], srows, sem0)
        ct = pltpu.async_copy(
            table_hbm.at[didx.at[pl.ds(b * _B, _B)]], trows, sem1)
        cs.wait()
        ct.wait()

        def group(g, bacc):
            for e in range(16):
                row = g * 16 + e
                for j in range(_D // _L):
                    bacc = bacc + (srows[row, pl.ds(j * _L, _L)]
                                   * trows[row, pl.ds(j * _L, _L)])
            return bacc

        return acc + lax.fori_loop(0, _G, group, jnp.zeros((_L,), jnp.float32))

    acc = lax.fori_loop(0, _NB, neg_batch, jnp.zeros((_L,), jnp.float32))
    accbuf[...] = acc
    pltpu.sync_copy(accbuf, nout_hbm.at[pl.ds(wid * _L, _L)])


_sc_dots = functools.partial(
    pl.kernel,
    mesh=plsc.VectorSubcoreMesh(core_axis_name="c", subcore_axis_name="s"),
    out_type=(
        jax.ShapeDtypeStruct((_E * _L,), jnp.float32),
        jax.ShapeDtypeStruct((_NW * _L,), jnp.float32),
    ),
    scratch_types=[
        pltpu.VMEM((_EPT,), jnp.int32),
        pltpu.VMEM((_EPT,), jnp.int32),
        pltpu.VMEM((_B, _D), jnp.float32),
        pltpu.VMEM((_B, _D), jnp.float32),
        pltpu.VMEM((_B * _L,), jnp.float32),
        pltpu.VMEM((_L,), jnp.float32),
        pltpu.SemaphoreType.DMA,
        pltpu.SemaphoreType.DMA,
    ],
)(_sc_body)


_ROWS = _E * _L // _D
_BLK = 2000
_NSTEP = _ROWS // _BLK


def _loss_body(pref, nref, oref):
    i = pl.program_id(0)
    part = pref[...]
    k = lax.broadcasted_iota(jnp.int32, (_D, _D), 0)
    j = lax.broadcasted_iota(jnp.int32, (_D, _D), 1)
    seg = jnp.where((j < _D // _L) & (k // _L == j), 1.0, 0.0).astype(jnp.float32)
    dots = jnp.dot(part, seg, preferred_element_type=jnp.float32)
    col = lax.broadcasted_iota(jnp.int32, (_BLK, _D), 1)
    terms = jnp.where(col < _D // _L,
                      jnp.log(1.0 - jnp.exp(-_EPS - dots)), 0.0)
    s = jnp.sum(terms)

    @pl.when(i == 0)
    def _():
        oref[...] = jnp.zeros_like(oref)

    oref[...] = oref[...] + jnp.reshape(s, (1, 1))

    @pl.when(i == _NSTEP - 1)
    def _():
        pos_sum = oref[0, 0]
        neg_sum = jnp.sum(nref[...])
        loss = -pos_sum / _E + neg_sum / _E
        oref[...] = jnp.reshape(loss, (1, 1))


def kernel(block_outputs, pos_edge_index, neg_edge_index):
    pos_i = pos_edge_index.astype(jnp.int32)
    neg_i = neg_edge_index.astype(jnp.int32)
    partials, negacc = _sc_dots(
        block_outputs, pos_i[0], pos_i[1], neg_i[0], neg_i[1])
    loss = pl.pallas_call(
        _loss_body,
        grid=(_NSTEP,),
        in_specs=[
            pl.BlockSpec((_BLK, _D), lambda i: (i, 0)),
            pl.BlockSpec((_NW * _L // _D, _D), lambda i: (0, 0)),
        ],
        out_specs=pl.BlockSpec((1, 1), lambda i: (0, 0)),
        out_shape=jax.ShapeDtypeStruct((1, 1), jnp.float32),
    )(partials.reshape(_ROWS, _D), negacc.reshape(_NW * _L // _D, _D))
    return loss[0, 0]
